# baseline (speedup 1.0000x reference)
import functools
import math

import jax
import jax.numpy as jnp
from jax.experimental import pallas as pl

H = 16
EPS = 1e-5
THETA = 10000.0
SCALE = 1.0
F32 = jnp.float32
NEG = -1e30


def _rms(x, g):
    return x * jax.lax.rsqrt(jnp.mean(x * x, axis=-1, keepdims=True) + EPS) * g


def _rmsnorm_mm_kern(x_ref, g_ref, w_ref, o_ref):
    h = _rms(x_ref[...], g_ref[...])
    o_ref[...] = jnp.dot(h, w_ref[...], preferred_element_type=F32)


def rmsnorm_mm(x, g, w, bm):
    S, D = x.shape
    N = w.shape[1]
    return pl.pallas_call(
        _rmsnorm_mm_kern,
        grid=(S // bm,),
        in_specs=[
            pl.BlockSpec((bm, D), lambda i: (i, 0)),
            pl.BlockSpec((1, D), lambda i: (0, 0)),
            pl.BlockSpec((D, N), lambda i: (0, 0)),
        ],
        out_specs=pl.BlockSpec((bm, N), lambda i: (i, 0)),
        out_shape=jax.ShapeDtypeStruct((S, N), F32),
    )(x, g.reshape(1, D), w)


def _attn_kern(q_ref, k_ref, v_ref, cq_ref, sq_ref, ck_ref, sk_ref,
               dq_ref, dk_ref, o_ref, *, bq, A):
    half = A // 2

    def rope(x, c, s):
        x1 = x[:, :half]
        x2 = x[:, half:]
        return jnp.concatenate([x1 * c + x2 * s, -x1 * s + x2 * c], axis=-1)

    q = rope(q_ref[0], cq_ref[...], sq_ref[...])
    k = rope(k_ref[0], ck_ref[...], sk_ref[...])
    s = jax.lax.dot_general(q, k, (((1,), (1,)), ((), ())),
                            preferred_element_type=F32)
    s = s * (1.0 / math.sqrt(A))
    Sk = k.shape[0]
    qi = pl.program_id(1)
    qpos = qi * bq + jax.lax.broadcasted_iota(jnp.int32, (bq, 1), 0)
    kpos = jax.lax.broadcasted_iota(jnp.int32, (1, Sk), 1)
    mask = (kpos <= qpos) & (dq_ref[...] == dk_ref[...])
    s = jnp.where(mask, s, NEG)
    m = jnp.max(s, axis=-1, keepdims=True)
    p = jnp.exp(s - m)
    p = p / jnp.sum(p, axis=-1, keepdims=True)
    o_ref[0] = jnp.dot(p, v_ref[0], preferred_element_type=F32)


def attention(qkv, cos, sin, doc_col, doc_row, bq):
    S = qkv.shape[0]
    D = qkv.shape[1] // 3
    A = D // H
    half = A // 2
    qh = qkv[:, :D].reshape(S, H, A).transpose(1, 0, 2)
    kh = qkv[:, D:2 * D].reshape(S, H, A).transpose(1, 0, 2)
    vh = qkv[:, 2 * D:].reshape(S, H, A).transpose(1, 0, 2)
    kern = functools.partial(_attn_kern, bq=bq, A=A)
    out = pl.pallas_call(
        kern,
        grid=(H, S // bq),
        in_specs=[
            pl.BlockSpec((1, bq, A), lambda h, i: (h, i, 0)),
            pl.BlockSpec((1, S, A), lambda h, i: (h, 0, 0)),
            pl.BlockSpec((1, S, A), lambda h, i: (h, 0, 0)),
            pl.BlockSpec((bq, half), lambda h, i: (i, 0)),
            pl.BlockSpec((bq, half), lambda h, i: (i, 0)),
            pl.BlockSpec((S, half), lambda h, i: (0, 0)),
            pl.BlockSpec((S, half), lambda h, i: (0, 0)),
            pl.BlockSpec((bq, 1), lambda h, i: (i, 0)),
            pl.BlockSpec((1, S), lambda h, i: (0, 0)),
        ],
        out_specs=pl.BlockSpec((1, bq, A), lambda h, i: (h, i, 0)),
        out_shape=jax.ShapeDtypeStruct((H, S, A), F32),
    )(qh, kh, vh, cos, sin, cos, sin, doc_col, doc_row)
    return out.transpose(1, 0, 2).reshape(S, D)


def _mm_add_kern(a_ref, w_ref, r_ref, o_ref):
    o_ref[...] = jnp.dot(a_ref[...], w_ref[...],
                         preferred_element_type=F32) + r_ref[...]


def mm_add(a, w, res, bm):
    S, K = a.shape
    N = w.shape[1]
    return pl.pallas_call(
        _mm_add_kern,
        grid=(S // bm,),
        in_specs=[
            pl.BlockSpec((bm, K), lambda i: (i, 0)),
            pl.BlockSpec((K, N), lambda i: (0, 0)),
            pl.BlockSpec((bm, N), lambda i: (i, 0)),
        ],
        out_specs=pl.BlockSpec((bm, N), lambda i: (i, 0)),
        out_shape=jax.ShapeDtypeStruct((S, N), F32),
    )(a, w, res)


def _ffn_kern(x_ref, g_ref, wu_ref, wd_ref, y_ref, hf_ref):
    x = x_ref[...]
    h = _rms(x, g_ref[...])
    hf_ref[...] = h
    u = jnp.dot(h, wu_ref[...], preferred_element_type=F32)
    F = wd_ref.shape[0]
    a1 = u[:, :F]
    a2 = u[:, F:]
    gated = a1 * jax.nn.sigmoid(a1) * a2
    y_ref[...] = jnp.dot(gated, wd_ref[...],
                         preferred_element_type=F32) + x


def ffn(x2, g, wup, wdn, bm):
    S, D = x2.shape
    N = wup.shape[1]
    F = wdn.shape[0]
    return pl.pallas_call(
        _ffn_kern,
        grid=(S // bm,),
        in_specs=[
            pl.BlockSpec((bm, D), lambda i: (i, 0)),
            pl.BlockSpec((1, D), lambda i: (0, 0)),
            pl.BlockSpec((D, N), lambda i: (0, 0)),
            pl.BlockSpec((F, D), lambda i: (0, 0)),
        ],
        out_specs=[
            pl.BlockSpec((bm, D), lambda i: (i, 0)),
            pl.BlockSpec((bm, D), lambda i: (i, 0)),
        ],
        out_shape=[
            jax.ShapeDtypeStruct((S, D), F32),
            jax.ShapeDtypeStruct((S, D), F32),
        ],
    )(x2, g.reshape(1, D), wup, wdn)


def _mm_kern(a_ref, w_ref, o_ref):
    o_ref[...] = jnp.dot(a_ref[...], w_ref[...], preferred_element_type=F32)


def mm_plain(a, w):
    S, K = a.shape
    N = w.shape[1]
    return pl.pallas_call(
        _mm_kern,
        in_specs=[pl.BlockSpec((S, K), lambda: (0, 0)),
                  pl.BlockSpec((K, N), lambda: (0, 0))],
        out_specs=pl.BlockSpec((S, N), lambda: (0, 0)),
        out_shape=jax.ShapeDtypeStruct((S, N), F32),
    )(a, w)


def _router_kern(tv_ref, idx_ref, val_ref, rb_ref, comb_ref, *, LE, TK):
    T = tv_ref.shape[0]
    tv = tv_ref[:, :LE]
    idx = idx_ref[...]
    val = val_ref[...]
    rb = rb_ref[...]
    lanes = jax.lax.broadcasted_iota(jnp.int32, (T, LE), 1)
    num = jnp.zeros((T, LE), F32)
    den = jnp.zeros((T, 1), F32)
    for kk in range(TK):
        oh = (idx[:, kk:kk + 1] == lanes).astype(F32)
        tvk = jnp.sum(tv * oh, axis=-1, keepdims=True)
        rbk = jnp.sum(rb * oh, axis=-1, keepdims=True)
        s = jax.nn.sigmoid(val[:, kk:kk + 1] + tvk + rbk)
        num = num + oh * s
        den = den + s
    comb_ref[...] = num / den * SCALE


def router(tv, idx, val, rbias):
    T = tv.shape[0]
    LE = rbias.shape[0]
    TK = idx.shape[1]
    kern = functools.partial(_router_kern, LE=LE, TK=TK)
    return pl.pallas_call(
        kern,
        in_specs=[
            pl.BlockSpec(tv.shape, lambda: (0, 0)),
            pl.BlockSpec((T, TK), lambda: (0, 0)),
            pl.BlockSpec((T, TK), lambda: (0, 0)),
            pl.BlockSpec((1, LE), lambda: (0, 0)),
        ],
        out_specs=pl.BlockSpec((T, LE), lambda: (0, 0)),
        out_shape=jax.ShapeDtypeStruct((T, LE), F32),
    )(tv, idx.astype(jnp.int32), val, rbias.reshape(1, LE))


def _moe_kern(hf_ref, w1_ref, w2_ref, w3_ref, comb_ref, res_ref, o_ref, *, LE):
    e = pl.program_id(0)
    h = hf_ref[...]
    lanes = jax.lax.broadcasted_iota(jnp.int32, (1, LE), 1)
    c = jnp.sum(comb_ref[...] * (lanes == e).astype(F32),
                axis=-1, keepdims=True)
    h1 = jnp.dot(h, w1_ref[0], preferred_element_type=F32)
    h2 = jnp.dot(h, w2_ref[0], preferred_element_type=F32)
    hh = h1 * jax.nn.sigmoid(h1) * h2
    yo = jnp.dot(hh, w3_ref[0], preferred_element_type=F32)

    @pl.when(e == 0)
    def _():
        o_ref[...] = res_ref[...]

    o_ref[...] += c * yo


def moe_experts(hf, w1, w2, w3t, comb, res):
    S, D = hf.shape
    LE = w1.shape[0]
    DE = w1.shape[2]
    kern = functools.partial(_moe_kern, LE=LE)
    return pl.pallas_call(
        kern,
        grid=(LE,),
        in_specs=[
            pl.BlockSpec((S, D), lambda e: (0, 0)),
            pl.BlockSpec((1, D, DE), lambda e: (e, 0, 0)),
            pl.BlockSpec((1, D, DE), lambda e: (e, 0, 0)),
            pl.BlockSpec((1, DE, D), lambda e: (e, 0, 0)),
            pl.BlockSpec((S, LE), lambda e: (0, 0)),
            pl.BlockSpec((S, D), lambda e: (0, 0)),
        ],
        out_specs=pl.BlockSpec((S, D), lambda e: (0, 0)),
        out_shape=jax.ShapeDtypeStruct((S, D), F32),
    )(hf, w1, w2, w3t, comb, res)


def _dense_layer(x, rope_doc, wqkv, wo, wup, wdn, g1, g2, bm, bq):
    cos, sin, doc_col, doc_row = rope_doc
    qkv = rmsnorm_mm(x, g1, wqkv, bm)
    xa = attention(qkv, cos, sin, doc_col, doc_row, bq)
    x2 = mm_add(xa, wo, x, bm)
    y, _ = ffn(x2, g2, wup, wdn, bm)
    return y


def _moe_layer(x, rope_doc, idx, val, wqkv, wo, g1, g2, w1, w2, w3t,
               tkeys_pad, rbias, wup, wdn, bm, bq):
    cos, sin, doc_col, doc_row = rope_doc
    qkv = rmsnorm_mm(x, g1, wqkv, bm)
    xa = attention(qkv, cos, sin, doc_col, doc_row, bq)
    x2 = mm_add(xa, wo, x, bm)
    y_sh, hf = ffn(x2, g2, wup, wdn, bm)
    tv = mm_plain(hf, tkeys_pad)
    comb = router(tv, idx, val, rbias)
    return moe_experts(hf, w1, w2, w3t, comb, res=y_sh)


def kernel(x, doc, indices, values, dl_attn_w, dl_attn_o_w, dl_ffn_up_w,
           dl_ffn_down_w, dl_attn_norm, dl_ffn_norm, ml_attn_w, ml_attn_o_w,
           ml_attn_norm, ml_ffn_norm, ml_experts, ml_token_keys,
           ml_router_bias, ml_ffn_up_w, ml_ffn_down_w):
    B, S, D = x.shape
    A = D // H
    bm = min(256, S)
    bq = min(256, S)

    inv = (1.0 / THETA) ** (jnp.arange(0, A, 2, dtype=F32) / A)
    fr = jnp.outer(jnp.arange(S, dtype=F32), inv)
    cos, sin = jnp.cos(fr), jnp.sin(fr)
    doc_col = doc.reshape(S, 1).astype(jnp.int32)
    doc_row = doc.reshape(1, S).astype(jnp.int32)
    rope_doc = (cos, sin, doc_col, doc_row)

    LE = ml_router_bias.shape[1]
    pad_to = max(128, LE)
    xs = x.reshape(S, D)

    xs = _dense_layer(xs, rope_doc, dl_attn_w[0], dl_attn_o_w[0],
                      dl_ffn_up_w[0], dl_ffn_down_w[0], dl_attn_norm[0],
                      dl_ffn_norm[0], bm, bq)

    L = ml_attn_w.shape[0]
    for j in range(L):
        tkeys_pad = jnp.pad(ml_token_keys[j], ((0, 0), (0, pad_to - LE)))
        w3t = ml_experts[j, 2].transpose(0, 2, 1)
        xs = _moe_layer(xs, rope_doc, indices[j], values[j], ml_attn_w[j],
                        ml_attn_o_w[j], ml_attn_norm[j], ml_ffn_norm[j],
                        ml_experts[j, 0], ml_experts[j, 1], w3t,
                        tkeys_pad, ml_router_bias[j], ml_ffn_up_w[j],
                        ml_ffn_down_w[j], bm, bq)

    xs = _dense_layer(xs, rope_doc, dl_attn_w[1], dl_attn_o_w[1],
                      dl_ffn_up_w[1], dl_ffn_down_w[1], dl_attn_norm[1],
                      dl_ffn_norm[1], bm, bq)
    return xs.reshape(B, S, D)


# speedup vs baseline: 1.0202x; 1.0202x over previous
import functools
import math

import jax
import jax.numpy as jnp
from jax.experimental import pallas as pl

H = 16
EPS = 1e-5
THETA = 10000.0
SCALE = 1.0
F32 = jnp.float32
NEG = -1e30


BF16 = jnp.bfloat16


def _dot(a, b):
    return jnp.dot(a.astype(BF16), b.astype(BF16),
                   preferred_element_type=F32)

def _rms(x, g):
    return x * jax.lax.rsqrt(jnp.mean(x * x, axis=-1, keepdims=True) + EPS) * g


def _rmsnorm_mm_kern(x_ref, g_ref, w_ref, o_ref):
    h = _rms(x_ref[...], g_ref[...])
    o_ref[...] = _dot(h, w_ref[...])


def rmsnorm_mm(x, g, w, bm):
    S, D = x.shape
    N = w.shape[1]
    return pl.pallas_call(
        _rmsnorm_mm_kern,
        grid=(S // bm,),
        in_specs=[
            pl.BlockSpec((bm, D), lambda i: (i, 0)),
            pl.BlockSpec((1, D), lambda i: (0, 0)),
            pl.BlockSpec((D, N), lambda i: (0, 0)),
        ],
        out_specs=pl.BlockSpec((bm, N), lambda i: (i, 0)),
        out_shape=jax.ShapeDtypeStruct((S, N), F32),
    )(x, g.reshape(1, D), w)


def _attn_kern(q_ref, k_ref, v_ref, cq_ref, sq_ref, ck_ref, sk_ref,
               dq_ref, dk_ref, o_ref, *, bq, A):
    half = A // 2

    def rope(x, c, s):
        x1 = x[:, :half]
        x2 = x[:, half:]
        return jnp.concatenate([x1 * c + x2 * s, -x1 * s + x2 * c], axis=-1)

    q = rope(q_ref[0], cq_ref[...], sq_ref[...])
    k = rope(k_ref[0], ck_ref[...], sk_ref[...])
    s = jax.lax.dot_general(q.astype(BF16), k.astype(BF16),
                            (((1,), (1,)), ((), ())),
                            preferred_element_type=F32)
    s = s * (1.0 / math.sqrt(A))
    Sk = k.shape[0]
    qi = pl.program_id(1)
    qpos = qi * bq + jax.lax.broadcasted_iota(jnp.int32, (bq, 1), 0)
    kpos = jax.lax.broadcasted_iota(jnp.int32, (1, Sk), 1)
    mask = (kpos <= qpos) & (dq_ref[...] == dk_ref[...])
    s = jnp.where(mask, s, NEG)
    m = jnp.max(s, axis=-1, keepdims=True)
    p = jnp.exp(s - m)
    p = p / jnp.sum(p, axis=-1, keepdims=True)
    o_ref[0] = _dot(p, v_ref[0])


def attention(qkv, cos, sin, doc_col, doc_row, bq):
    S = qkv.shape[0]
    D = qkv.shape[1] // 3
    A = D // H
    half = A // 2
    qh = qkv[:, :D].reshape(S, H, A).transpose(1, 0, 2)
    kh = qkv[:, D:2 * D].reshape(S, H, A).transpose(1, 0, 2)
    vh = qkv[:, 2 * D:].reshape(S, H, A).transpose(1, 0, 2)
    kern = functools.partial(_attn_kern, bq=bq, A=A)
    out = pl.pallas_call(
        kern,
        grid=(H, S // bq),
        in_specs=[
            pl.BlockSpec((1, bq, A), lambda h, i: (h, i, 0)),
            pl.BlockSpec((1, S, A), lambda h, i: (h, 0, 0)),
            pl.BlockSpec((1, S, A), lambda h, i: (h, 0, 0)),
            pl.BlockSpec((bq, half), lambda h, i: (i, 0)),
            pl.BlockSpec((bq, half), lambda h, i: (i, 0)),
            pl.BlockSpec((S, half), lambda h, i: (0, 0)),
            pl.BlockSpec((S, half), lambda h, i: (0, 0)),
            pl.BlockSpec((bq, 1), lambda h, i: (i, 0)),
            pl.BlockSpec((1, S), lambda h, i: (0, 0)),
        ],
        out_specs=pl.BlockSpec((1, bq, A), lambda h, i: (h, i, 0)),
        out_shape=jax.ShapeDtypeStruct((H, S, A), F32),
    )(qh, kh, vh, cos, sin, cos, sin, doc_col, doc_row)
    return out.transpose(1, 0, 2).reshape(S, D)


def _mm_add_kern(a_ref, w_ref, r_ref, o_ref):
    o_ref[...] = _dot(a_ref[...], w_ref[...]) + r_ref[...]


def mm_add(a, w, res, bm):
    S, K = a.shape
    N = w.shape[1]
    return pl.pallas_call(
        _mm_add_kern,
        grid=(S // bm,),
        in_specs=[
            pl.BlockSpec((bm, K), lambda i: (i, 0)),
            pl.BlockSpec((K, N), lambda i: (0, 0)),
            pl.BlockSpec((bm, N), lambda i: (i, 0)),
        ],
        out_specs=pl.BlockSpec((bm, N), lambda i: (i, 0)),
        out_shape=jax.ShapeDtypeStruct((S, N), F32),
    )(a, w, res)


def _ffn_kern(x_ref, g_ref, wu_ref, wd_ref, y_ref, hf_ref):
    x = x_ref[...]
    h = _rms(x, g_ref[...])
    hf_ref[...] = h
    u = _dot(h, wu_ref[...])
    F = wd_ref.shape[0]
    a1 = u[:, :F]
    a2 = u[:, F:]
    gated = a1 * jax.nn.sigmoid(a1) * a2
    y_ref[...] = _dot(gated, wd_ref[...]) + x


def ffn(x2, g, wup, wdn, bm):
    S, D = x2.shape
    N = wup.shape[1]
    F = wdn.shape[0]
    return pl.pallas_call(
        _ffn_kern,
        grid=(S // bm,),
        in_specs=[
            pl.BlockSpec((bm, D), lambda i: (i, 0)),
            pl.BlockSpec((1, D), lambda i: (0, 0)),
            pl.BlockSpec((D, N), lambda i: (0, 0)),
            pl.BlockSpec((F, D), lambda i: (0, 0)),
        ],
        out_specs=[
            pl.BlockSpec((bm, D), lambda i: (i, 0)),
            pl.BlockSpec((bm, D), lambda i: (i, 0)),
        ],
        out_shape=[
            jax.ShapeDtypeStruct((S, D), F32),
            jax.ShapeDtypeStruct((S, D), F32),
        ],
    )(x2, g.reshape(1, D), wup, wdn)


def _mm_kern(a_ref, w_ref, o_ref):
    o_ref[...] = _dot(a_ref[...], w_ref[...])


def mm_plain(a, w):
    S, K = a.shape
    N = w.shape[1]
    return pl.pallas_call(
        _mm_kern,
        in_specs=[pl.BlockSpec((S, K), lambda: (0, 0)),
                  pl.BlockSpec((K, N), lambda: (0, 0))],
        out_specs=pl.BlockSpec((S, N), lambda: (0, 0)),
        out_shape=jax.ShapeDtypeStruct((S, N), F32),
    )(a, w)


def _router_kern(tv_ref, idx_ref, val_ref, rb_ref, comb_ref, *, LE, TK):
    T = tv_ref.shape[0]
    tv = tv_ref[:, :LE]
    idx = idx_ref[...]
    val = val_ref[...]
    rb = rb_ref[...]
    lanes = jax.lax.broadcasted_iota(jnp.int32, (T, LE), 1)
    num = jnp.zeros((T, LE), F32)
    den = jnp.zeros((T, 1), F32)
    for kk in range(TK):
        oh = (idx[:, kk:kk + 1] == lanes).astype(F32)
        tvk = jnp.sum(tv * oh, axis=-1, keepdims=True)
        rbk = jnp.sum(rb * oh, axis=-1, keepdims=True)
        s = jax.nn.sigmoid(val[:, kk:kk + 1] + tvk + rbk)
        num = num + oh * s
        den = den + s
    comb_ref[...] = num / den * SCALE


def router(tv, idx, val, rbias):
    T = tv.shape[0]
    LE = rbias.shape[0]
    TK = idx.shape[1]
    kern = functools.partial(_router_kern, LE=LE, TK=TK)
    return pl.pallas_call(
        kern,
        in_specs=[
            pl.BlockSpec(tv.shape, lambda: (0, 0)),
            pl.BlockSpec((T, TK), lambda: (0, 0)),
            pl.BlockSpec((T, TK), lambda: (0, 0)),
            pl.BlockSpec((1, LE), lambda: (0, 0)),
        ],
        out_specs=pl.BlockSpec((T, LE), lambda: (0, 0)),
        out_shape=jax.ShapeDtypeStruct((T, LE), F32),
    )(tv, idx.astype(jnp.int32), val, rbias.reshape(1, LE))


def _moe_kern(hf_ref, w1_ref, w2_ref, w3_ref, comb_ref, res_ref, o_ref, *, LE):
    e = pl.program_id(0)
    h = hf_ref[...]
    lanes = jax.lax.broadcasted_iota(jnp.int32, (1, LE), 1)
    c = jnp.sum(comb_ref[...] * (lanes == e).astype(F32),
                axis=-1, keepdims=True)
    h1 = _dot(h, w1_ref[0])
    h2 = _dot(h, w2_ref[0])
    hh = h1 * jax.nn.sigmoid(h1) * h2
    yo = _dot(hh, w3_ref[0])

    @pl.when(e == 0)
    def _():
        o_ref[...] = res_ref[...]

    o_ref[...] += c * yo


def moe_experts(hf, w1, w2, w3t, comb, res):
    S, D = hf.shape
    LE = w1.shape[0]
    DE = w1.shape[2]
    kern = functools.partial(_moe_kern, LE=LE)
    return pl.pallas_call(
        kern,
        grid=(LE,),
        in_specs=[
            pl.BlockSpec((S, D), lambda e: (0, 0)),
            pl.BlockSpec((1, D, DE), lambda e: (e, 0, 0)),
            pl.BlockSpec((1, D, DE), lambda e: (e, 0, 0)),
            pl.BlockSpec((1, DE, D), lambda e: (e, 0, 0)),
            pl.BlockSpec((S, LE), lambda e: (0, 0)),
            pl.BlockSpec((S, D), lambda e: (0, 0)),
        ],
        out_specs=pl.BlockSpec((S, D), lambda e: (0, 0)),
        out_shape=jax.ShapeDtypeStruct((S, D), F32),
    )(hf, w1, w2, w3t, comb, res)


def _dense_layer(x, rope_doc, wqkv, wo, wup, wdn, g1, g2, bm, bq):
    cos, sin, doc_col, doc_row = rope_doc
    qkv = rmsnorm_mm(x, g1, wqkv, bm)
    xa = attention(qkv, cos, sin, doc_col, doc_row, bq)
    x2 = mm_add(xa, wo, x, bm)
    y, _ = ffn(x2, g2, wup, wdn, bm)
    return y


def _moe_layer(x, rope_doc, idx, val, wqkv, wo, g1, g2, w1, w2, w3t,
               tkeys_pad, rbias, wup, wdn, bm, bq):
    cos, sin, doc_col, doc_row = rope_doc
    qkv = rmsnorm_mm(x, g1, wqkv, bm)
    xa = attention(qkv, cos, sin, doc_col, doc_row, bq)
    x2 = mm_add(xa, wo, x, bm)
    y_sh, hf = ffn(x2, g2, wup, wdn, bm)
    tv = mm_plain(hf, tkeys_pad)
    comb = router(tv, idx, val, rbias)
    return moe_experts(hf, w1, w2, w3t, comb, res=y_sh)


def kernel(x, doc, indices, values, dl_attn_w, dl_attn_o_w, dl_ffn_up_w,
           dl_ffn_down_w, dl_attn_norm, dl_ffn_norm, ml_attn_w, ml_attn_o_w,
           ml_attn_norm, ml_ffn_norm, ml_experts, ml_token_keys,
           ml_router_bias, ml_ffn_up_w, ml_ffn_down_w):
    B, S, D = x.shape
    A = D // H
    bm = min(256, S)
    bq = min(256, S)

    inv = (1.0 / THETA) ** (jnp.arange(0, A, 2, dtype=F32) / A)
    fr = jnp.outer(jnp.arange(S, dtype=F32), inv)
    cos, sin = jnp.cos(fr), jnp.sin(fr)
    doc_col = doc.reshape(S, 1).astype(jnp.int32)
    doc_row = doc.reshape(1, S).astype(jnp.int32)
    rope_doc = (cos, sin, doc_col, doc_row)

    LE = ml_router_bias.shape[1]
    pad_to = max(128, LE)
    xs = x.reshape(S, D)

    xs = _dense_layer(xs, rope_doc, dl_attn_w[0], dl_attn_o_w[0],
                      dl_ffn_up_w[0], dl_ffn_down_w[0], dl_attn_norm[0],
                      dl_ffn_norm[0], bm, bq)

    L = ml_attn_w.shape[0]
    for j in range(L):
        tkeys_pad = jnp.pad(ml_token_keys[j], ((0, 0), (0, pad_to - LE)))
        w3t = ml_experts[j, 2].transpose(0, 2, 1)
        xs = _moe_layer(xs, rope_doc, indices[j], values[j], ml_attn_w[j],
                        ml_attn_o_w[j], ml_attn_norm[j], ml_ffn_norm[j],
                        ml_experts[j, 0], ml_experts[j, 1], w3t,
                        tkeys_pad, ml_router_bias[j], ml_ffn_up_w[j],
                        ml_ffn_down_w[j], bm, bq)

    xs = _dense_layer(xs, rope_doc, dl_attn_w[1], dl_attn_o_w[1],
                      dl_ffn_up_w[1], dl_ffn_down_w[1], dl_attn_norm[1],
                      dl_ffn_norm[1], bm, bq)
    return xs.reshape(B, S, D)
